# baseline (speedup 1.0000x reference)
import functools

import jax
import jax.numpy as jnp
from jax import lax
from jax.experimental import pallas as pl
from jax.experimental.pallas import tpu as pltpu
from jax.experimental.pallas import tpu_sc as plsc

_NUM_WORKERS = 32


def kernel(distance, similarities):
    B, I, J, K, D = distance.shape
    N = B * I * J
    PW = N // _NUM_WORKERS
    assert PW * _NUM_WORKERS == N
    NCHUNK = 3
    CR = PW // NCHUNK
    WPB = _NUM_WORKERS // B
    IROWS = I // WPB
    RPC = IROWS // NCHUNK

    dist = distance.reshape(N * K, D)
    sims = similarities.reshape(N, K)

    NB = N // B

    NQ = 2
    NR = N // NQ

    def argmax_body(s_hbm, arg_ref, idx_ref, s_v, sems):
        copies = [
            pltpu.async_copy(
                s_hbm.at[pl.ds(q * NR, NR)], s_v.at[pl.ds(q * NR, NR)], sems.at[q]
            )
            for q in range(NQ)
        ]
        for q in range(NQ):
            copies[q].wait()
            s = s_v[pl.ds(q * NR, NR), :]
            mx = jnp.max(s, axis=1, keepdims=True)
            kio = lax.broadcasted_iota(jnp.int32, (NR, K), 1).astype(jnp.float32)
            bf = jnp.min(jnp.where(s == mx, kio, float(K)), axis=1)
            bi = bf.astype(jnp.int32)
            arg_ref[pl.ds(q * (B // NQ), B // NQ)] = bi.reshape(B // NQ, I, J)
            idx_ref[pl.ds(q * NR, NR)] = (
                q * NR + lax.iota(jnp.int32, NR)
            ) * K + bi

    arg, idx = pl.pallas_call(
        argmax_body,
        in_specs=[pl.BlockSpec(memory_space=pl.ANY)],
        scratch_shapes=[
            pltpu.VMEM((N, K), jnp.float32),
            pltpu.SemaphoreType.DMA((NQ,)),
        ],
        out_shape=[
            jax.ShapeDtypeStruct((B, I, J), jnp.int32),
            jax.ShapeDtypeStruct((N,), jnp.int32),
        ],
    )(sims)

    mesh = plsc.VectorSubcoreMesh(core_axis_name="c", subcore_axis_name="s")

    @functools.partial(
        pl.kernel,
        mesh=mesh,
        compiler_params=pltpu.CompilerParams(needs_layout_passes=False),
        out_type=jax.ShapeDtypeStruct((B, I, J, D), jnp.float32),
        scratch_types=[
            pltpu.VMEM((PW,), jnp.int32),
            pltpu.VMEM((PW, D), jnp.float32),
            pltpu.SemaphoreType.DMA,
            pltpu.SemaphoreType.DMA,
        ],
    )
    def gather_body(dist_hbm, idx_hbm, out_hbm, idx_v, rows_v, sem_g, sem_w):
        wid = lax.axis_index("s") * 2 + lax.axis_index("c")
        base = wid * PW
        b0 = wid // WPB
        i0 = (wid % WPB) * IROWS
        pltpu.sync_copy(idx_hbm.at[pl.ds(base, PW)], idx_v)
        chunk_rows = (1, 2, 2, 1)
        offs = [sum(chunk_rows[:c]) for c in range(len(chunk_rows))]
        gathers = [
            pltpu.async_copy(
                dist_hbm.at[idx_v.at[pl.ds(o * J, cr * J)]],
                rows_v.at[pl.ds(o * J, cr * J)],
                sem_g,
            )
            for o, cr in zip(offs, chunk_rows)
        ]
        writes = []
        for c, (o, cr) in enumerate(zip(offs, chunk_rows)):
            gathers[c].wait()
            for r in range(cr):
                ri = o + r
                writes.append(
                    pltpu.async_copy(
                        rows_v.at[pl.ds(ri * J, J)], out_hbm.at[b0, i0 + ri], sem_w
                    )
                )
        for w in writes:
            w.wait()

    out = gather_body(dist, idx)
    return out, arg


# speedup vs baseline: 1.0185x; 1.0185x over previous
import functools

import jax
import jax.numpy as jnp
from jax import lax
from jax.experimental import pallas as pl
from jax.experimental.pallas import tpu as pltpu
from jax.experimental.pallas import tpu_sc as plsc

_NUM_WORKERS = 32


def kernel(distance, similarities):
    B, I, J, K, D = distance.shape
    N = B * I * J
    PW = N // _NUM_WORKERS
    assert PW * _NUM_WORKERS == N
    NCHUNK = 3
    CR = PW // NCHUNK
    WPB = _NUM_WORKERS // B
    IROWS = I // WPB
    RPC = IROWS // NCHUNK

    dist = distance.reshape(N * K, D)
    sims = similarities.reshape(N, K)

    NB = N // B

    NQ = 2
    NR = N // NQ

    def argmax_body(s_hbm, arg_ref, idx_ref, s_v, sems):
        copies = [
            pltpu.async_copy(
                s_hbm.at[pl.ds(q * NR, NR)], s_v.at[pl.ds(q * NR, NR)], sems.at[q]
            )
            for q in range(NQ)
        ]
        for q in range(NQ):
            copies[q].wait()
            s = s_v[pl.ds(q * NR, NR), :]
            mx = jnp.max(s, axis=1, keepdims=True)
            kio = lax.broadcasted_iota(jnp.int32, (NR, K), 1).astype(jnp.float32)
            bf = jnp.min(jnp.where(s == mx, kio, float(K)), axis=1)
            bi = bf.astype(jnp.int32)
            arg_ref[pl.ds(q * (B // NQ), B // NQ)] = bi.reshape(B // NQ, I, J)
            idx_ref[pl.ds(q * NR, NR)] = (
                q * NR + lax.iota(jnp.int32, NR)
            ) * K + bi

    arg, idx = pl.pallas_call(
        argmax_body,
        in_specs=[pl.BlockSpec(memory_space=pl.ANY)],
        scratch_shapes=[
            pltpu.VMEM((N, K), jnp.float32),
            pltpu.SemaphoreType.DMA((NQ,)),
        ],
        out_shape=[
            jax.ShapeDtypeStruct((B, I, J), jnp.int32),
            jax.ShapeDtypeStruct((N,), jnp.int32),
        ],
    )(sims)

    mesh = plsc.VectorSubcoreMesh(core_axis_name="c", subcore_axis_name="s")

    @functools.partial(
        pl.kernel,
        mesh=mesh,
        compiler_params=pltpu.CompilerParams(needs_layout_passes=False),
        out_type=jax.ShapeDtypeStruct((B, I, J, D), jnp.float32),
        scratch_types=[
            pltpu.VMEM((PW,), jnp.int32),
            pltpu.VMEM((PW, D), jnp.float32),
            pltpu.SemaphoreType.DMA,
            pltpu.SemaphoreType.DMA,
        ],
    )
    def gather_body(dist_hbm, idx_hbm, out_hbm, idx_v, rows_v, sem_g, sem_w):
        wid = lax.axis_index("s") * 2 + lax.axis_index("c")
        base = wid * PW
        b0 = wid // WPB
        i0 = (wid % WPB) * IROWS
        pltpu.sync_copy(idx_hbm.at[pl.ds(base, PW)], idx_v)
        chunk_rows = (3, 2, 1)
        offs = [sum(chunk_rows[:c]) for c in range(len(chunk_rows))]
        gathers = [
            pltpu.async_copy(
                dist_hbm.at[idx_v.at[pl.ds(o * J, cr * J)]],
                rows_v.at[pl.ds(o * J, cr * J)],
                sem_g,
            )
            for o, cr in zip(offs, chunk_rows)
        ]
        writes = []
        for c, (o, cr) in enumerate(zip(offs, chunk_rows)):
            gathers[c].wait()
            for r in range(cr):
                ri = o + r
                writes.append(
                    pltpu.async_copy(
                        rows_v.at[pl.ds(ri * J, J)], out_hbm.at[b0, i0 + ri], sem_w
                    )
                )
        for w in writes:
            w.wait()

    out = gather_body(dist, idx)
    return out, arg
